# Initial kernel scaffold; baseline (speedup 1.0000x reference)
import jax
import jax.numpy as jnp
from jax.experimental import pallas as pl


def kernel(x, edge_index, W1l, b1, W1r, W2l, b2, W2r):
    raise NotImplementedError("write your pallas kernel here")


# speedup vs baseline: 7.6876x; 7.6876x over previous
import functools

import jax
import jax.numpy as jnp
from jax import lax
from jax.experimental import pallas as pl
from jax.experimental.pallas import tpu as pltpu
from jax.experimental.pallas import tpu_sc as plsc

N = 10000
E = 320000
D_IN = 128
D_H = 64
D_OUT = 128

NC = 2
NS = 16
NW = NC * NS
EPT = E // NW
CH = 128
NFULL = EPT // CH
TAIL = EPT - NFULL * CH
WPT = 624
WREM = N - NS * WPT
ZROWS = 156
DW = 16


def _seg_sum_kernel(with_deg: bool):
    mesh = plsc.VectorSubcoreMesh(core_axis_name="c", subcore_axis_name="s",
                                  num_cores=NC, num_subcores=NS)

    out_type = [jax.ShapeDtypeStruct((NC, N, D_H), jnp.float32)]
    if with_deg:
        out_type.append(jax.ShapeDtypeStruct((NC, N, DW), jnp.float32))

    scratch = [
        pltpu.VMEM((CH,), jnp.int32),
        pltpu.VMEM((CH,), jnp.int32),
        pltpu.VMEM((TAIL,), jnp.int32),
        pltpu.VMEM((TAIL,), jnp.int32),
        pltpu.VMEM((CH, D_H), jnp.float32),
        pltpu.VMEM((TAIL, D_H), jnp.float32),
        pltpu.VMEM((ZROWS, D_H), jnp.float32),
        pltpu.VMEM_SHARED((N, D_H), jnp.float32),
        pltpu.SemaphoreType.DMA,
    ]
    if with_deg:
        scratch += [
            pltpu.VMEM((CH, DW), jnp.float32),
            pltpu.VMEM_SHARED((N, DW), jnp.float32),
        ]

    def body(*refs):
        if with_deg:
            (src_h, dst_h, tab_h,
             part_h, degp_h,
             sidx, didx, sidx_t, didx_t, rows, rows_t, zbuf, acc, sem,
             ones_v, dacc) = refs
        else:
            (src_h, dst_h, tab_h,
             part_h,
             sidx, didx, sidx_t, didx_t, rows, rows_t, zbuf, acc, sem) = refs

        c = lax.axis_index("c")
        s = lax.axis_index("s")
        wid = s * NC + c
        ebase = pl.multiple_of(wid * EPT, 8)
        rbase = pl.multiple_of(s * WPT, 8)

        zero16 = jnp.zeros((16,), jnp.float32)

        def zrow(r, carry):
            for j in range(D_H // 16):
                zbuf[r, pl.ds(j * 16, 16)] = zero16
            return carry

        lax.fori_loop(0, ZROWS, zrow, 0)
        for k in range(WPT // ZROWS):
            pltpu.sync_copy(zbuf, acc.at[pl.ds(rbase + k * ZROWS, ZROWS)])
            if with_deg:
                pltpu.sync_copy(zbuf.at[pl.ds(0, ZROWS), pl.ds(0, DW)],
                                dacc.at[pl.ds(rbase + k * ZROWS, ZROWS)])

        @pl.when(s == NS - 1)
        def _():
            pltpu.sync_copy(zbuf.at[pl.ds(0, WREM)],
                            acc.at[pl.ds(NS * WPT, WREM)])
            if with_deg:
                pltpu.sync_copy(zbuf.at[pl.ds(0, WREM), pl.ds(0, DW)],
                                dacc.at[pl.ds(NS * WPT, WREM)])
        if with_deg:
            one16 = jnp.ones((16,), jnp.float32)

            def orow(r, carry):
                ones_v[r, pl.ds(0, DW)] = one16
                return carry

            lax.fori_loop(0, CH, orow, 0)

        plsc.subcore_barrier()

        def chunk(i, carry):
            off = pl.multiple_of(ebase + i * CH, 8)
            pltpu.sync_copy(src_h.at[pl.ds(off, CH)], sidx)
            pltpu.sync_copy(dst_h.at[pl.ds(off, CH)], didx)
            pltpu.async_copy(tab_h.at[sidx], rows, sem).wait()
            pltpu.sync_copy(rows, acc.at[didx], add=True)
            if with_deg:
                pltpu.sync_copy(ones_v, dacc.at[didx], add=True)
            return carry

        lax.fori_loop(0, NFULL, chunk, 0)

        offt = pl.multiple_of(ebase + NFULL * CH, 8)
        pltpu.sync_copy(src_h.at[pl.ds(offt, TAIL)], sidx_t)
        pltpu.sync_copy(dst_h.at[pl.ds(offt, TAIL)], didx_t)
        pltpu.async_copy(tab_h.at[sidx_t], rows_t, sem).wait()
        pltpu.sync_copy(rows_t, acc.at[didx_t], add=True)
        if with_deg:
            pltpu.sync_copy(ones_v.at[pl.ds(0, TAIL)], dacc.at[didx_t],
                            add=True)

        plsc.subcore_barrier()

        pltpu.sync_copy(acc.at[pl.ds(rbase, WPT)],
                        part_h.at[c, pl.ds(rbase, WPT)])

        if with_deg:
            pltpu.sync_copy(dacc.at[pl.ds(rbase, WPT)],
                            degp_h.at[c, pl.ds(rbase, WPT)])

        @pl.when(s == NS - 1)
        def _():
            pltpu.sync_copy(acc.at[pl.ds(NS * WPT, WREM)],
                            part_h.at[c, pl.ds(NS * WPT, WREM)])
            if with_deg:
                pltpu.sync_copy(dacc.at[pl.ds(NS * WPT, WREM)],
                                degp_h.at[c, pl.ds(NS * WPT, WREM)])

    ot = tuple(out_type) if with_deg else out_type[0]
    return pl.kernel(
        body, out_type=ot, mesh=mesh, scratch_types=scratch,
        compiler_params=pltpu.CompilerParams(use_tc_tiling_on_sc=False))


_seg_sum_deg = _seg_sum_kernel(True)
_seg_sum = _seg_sum_kernel(False)


BLK = 2000


def _tc1_body(x_ref, w1l_ref, w1r_ref, b1_ref, y1_ref, z1_ref):
    xb = x_ref[...]
    y1_ref[...] = jnp.dot(xb, w1l_ref[...], preferred_element_type=jnp.float32)
    z1_ref[...] = (jnp.dot(xb, w1r_ref[...],
                           preferred_element_type=jnp.float32) + b1_ref[...])


def _tc2_body(p_ref, dp_ref, z1_ref, w2r_ref, b2_ref, h_ref, z2_ref, rd_ref):
    rdeg = 1.0 / jnp.maximum(dp_ref[0, :, 0:1] + dp_ref[1, :, 0:1], 1.0)
    h = jnp.maximum((p_ref[0] + p_ref[1]) * rdeg + z1_ref[...], 0.0)
    rd_ref[...] = rdeg
    h_ref[...] = h
    z2_ref[...] = (jnp.dot(h, w2r_ref[...],
                           preferred_element_type=jnp.float32) + b2_ref[...])


def _tc3_body(p_ref, rd_ref, z2_ref, w2l_ref, o_ref):
    a2 = (p_ref[0] + p_ref[1]) * rd_ref[...]
    o_ref[...] = (jnp.dot(a2, w2l_ref[...],
                          preferred_element_type=jnp.float32) + z2_ref[...])


def _row_spec(d):
    return pl.BlockSpec((BLK, d), lambda i: (i, 0))


def _part_spec(d):
    return pl.BlockSpec((NC, BLK, d), lambda i: (0, i, 0))


def _full_spec(r, d):
    return pl.BlockSpec((r, d), lambda i: (0, 0))


_tc1 = pl.pallas_call(
    _tc1_body,
    grid=(N // BLK,),
    in_specs=[_row_spec(D_IN), _full_spec(D_IN, D_H), _full_spec(D_IN, D_H),
              _full_spec(1, D_H)],
    out_specs=[_row_spec(D_H), _row_spec(D_H)],
    out_shape=[jax.ShapeDtypeStruct((N, D_H), jnp.float32),
               jax.ShapeDtypeStruct((N, D_H), jnp.float32)],
)

_tc2 = pl.pallas_call(
    _tc2_body,
    grid=(N // BLK,),
    in_specs=[_part_spec(D_H), _part_spec(DW), _row_spec(D_H),
              _full_spec(D_H, D_OUT), _full_spec(1, D_OUT)],
    out_specs=[_row_spec(D_H), _row_spec(D_OUT), _row_spec(1)],
    out_shape=[jax.ShapeDtypeStruct((N, D_H), jnp.float32),
               jax.ShapeDtypeStruct((N, D_OUT), jnp.float32),
               jax.ShapeDtypeStruct((N, 1), jnp.float32)],
)

_tc3 = pl.pallas_call(
    _tc3_body,
    grid=(N // BLK,),
    in_specs=[_part_spec(D_H), _row_spec(1), _row_spec(D_OUT),
              _full_spec(D_H, D_OUT)],
    out_specs=_row_spec(D_OUT),
    out_shape=jax.ShapeDtypeStruct((N, D_OUT), jnp.float32),
)


def kernel(x, edge_index, W1l, b1, W1r, W2l, b2, W2r):
    src = edge_index[0]
    dst = edge_index[1]

    y1, z1 = _tc1(x, W1l.T, W1r.T, b1.reshape(1, D_H))
    part1, degp = _seg_sum_deg(src, dst, y1)
    h, z2, rdeg = _tc2(part1, degp, z1, W2r.T, b2.reshape(1, D_OUT))
    part2 = _seg_sum(src, dst, h)
    return _tc3(part2, rdeg, z2, W2l.T)


# speedup vs baseline: 15.1506x; 1.9708x over previous
import functools

import jax
import jax.numpy as jnp
from jax import lax
from jax.experimental import pallas as pl
from jax.experimental.pallas import tpu as pltpu
from jax.experimental.pallas import tpu_sc as plsc

N = 10000
E = 320000
D_IN = 128
D_H = 64
D_OUT = 128

NC = 2
NS = 16
NW = NC * NS
EPT = E // NW
CH = 128
NFULL = EPT // CH
TAIL = EPT - NFULL * CH
WPT = 624
WREM = N - NS * WPT
ZROWS = 156
DW = 16


def _seg_sum_kernel(with_deg: bool):
    mesh = plsc.VectorSubcoreMesh(core_axis_name="c", subcore_axis_name="s",
                                  num_cores=NC, num_subcores=NS)

    out_type = [jax.ShapeDtypeStruct((NC, N, D_H), jnp.float32)]
    if with_deg:
        out_type.append(jax.ShapeDtypeStruct((NC, N, DW), jnp.float32))

    scratch = [
        pltpu.VMEM((EPT,), jnp.int32),
        pltpu.VMEM((EPT,), jnp.int32),
        pltpu.VMEM((CH,), jnp.int32),
        pltpu.VMEM((CH,), jnp.int32),
        pltpu.VMEM((CH,), jnp.int32),
        pltpu.VMEM((CH,), jnp.int32),
        pltpu.VMEM((TAIL,), jnp.int32),
        pltpu.VMEM((TAIL,), jnp.int32),
        pltpu.VMEM((CH, D_H), jnp.float32),
        pltpu.VMEM((CH, D_H), jnp.float32),
        pltpu.VMEM((TAIL, D_H), jnp.float32),
        pltpu.VMEM((ZROWS, D_H), jnp.float32),
        pltpu.VMEM_SHARED((N, D_H), jnp.float32),
        pltpu.SemaphoreType.DMA,
        pltpu.SemaphoreType.DMA,
        pltpu.SemaphoreType.DMA,
        pltpu.SemaphoreType.DMA,
    ]
    if with_deg:
        scratch += [
            pltpu.VMEM((CH, DW), jnp.float32),
            pltpu.VMEM_SHARED((N, DW), jnp.float32),
            pltpu.SemaphoreType.DMA,
            pltpu.SemaphoreType.DMA,
        ]

    def body(*refs):
        if with_deg:
            (src_h, dst_h, tab_h,
             part_h, degp_h,
             srcall, dstall, sidx0, sidx1, didx0, didx1, sidx_t, didx_t,
             rows0, rows1, rows_t, zbuf, acc,
             gsem0, gsem1, ssem0, ssem1,
             ones_v, dacc, dsem0, dsem1) = refs
        else:
            (src_h, dst_h, tab_h,
             part_h,
             srcall, dstall, sidx0, sidx1, didx0, didx1, sidx_t, didx_t,
             rows0, rows1, rows_t, zbuf, acc,
             gsem0, gsem1, ssem0, ssem1) = refs
            dsem0 = dsem1 = None
        slots = ((sidx0, didx0, rows0, gsem0, ssem0, dsem0),
                 (sidx1, didx1, rows1, gsem1, ssem1, dsem1))

        c = lax.axis_index("c")
        s = lax.axis_index("s")
        wid = s * NC + c
        ebase = pl.multiple_of(wid * EPT, 8)
        rbase = pl.multiple_of(s * WPT, 8)

        zero16 = jnp.zeros((16,), jnp.float32)

        def zrow(r, carry):
            for j in range(D_H // 16):
                zbuf[r, pl.ds(j * 16, 16)] = zero16
            return carry

        lax.fori_loop(0, ZROWS, zrow, 0)
        for k in range(WPT // ZROWS):
            pltpu.sync_copy(zbuf, acc.at[pl.ds(rbase + k * ZROWS, ZROWS)])
            if with_deg:
                pltpu.sync_copy(zbuf.at[pl.ds(0, ZROWS), pl.ds(0, DW)],
                                dacc.at[pl.ds(rbase + k * ZROWS, ZROWS)])

        @pl.when(s == NS - 1)
        def _():
            pltpu.sync_copy(zbuf.at[pl.ds(0, WREM)],
                            acc.at[pl.ds(NS * WPT, WREM)])
            if with_deg:
                pltpu.sync_copy(zbuf.at[pl.ds(0, WREM), pl.ds(0, DW)],
                                dacc.at[pl.ds(NS * WPT, WREM)])
        if with_deg:
            one16 = jnp.ones((16,), jnp.float32)

            def orow(r, carry):
                ones_v[r, pl.ds(0, DW)] = one16
                return carry

            lax.fori_loop(0, CH, orow, 0)

        pltpu.sync_copy(src_h.at[pl.ds(ebase, EPT)], srcall)
        pltpu.sync_copy(dst_h.at[pl.ds(ebase, EPT)], dstall)

        plsc.subcore_barrier()

        def stage(sl, ci):
            sidx_k, didx_k = sl[0], sl[1]
            off = ci * CH
            for j in range(CH // 16):
                sidx_k[pl.ds(j * 16, 16)] = srcall[pl.ds(off + j * 16, 16)]
                didx_k[pl.ds(j * 16, 16)] = dstall[pl.ds(off + j * 16, 16)]

        def fire_g(sl):
            pltpu.async_copy(tab_h.at[sl[0]], sl[2], sl[3])

        def wait_g(sl):
            pltpu.make_async_copy(tab_h.at[sl[0]], sl[2], sl[3]).wait()

        def fire_s(sl):
            pltpu.async_copy(sl[2], acc.at[sl[1]], sl[4], add=True)
            if with_deg:
                pltpu.async_copy(ones_v, dacc.at[sl[1]], sl[5], add=True)

        def wait_s(sl):
            pltpu.make_async_copy(sl[2], acc.at[sl[1]], sl[4]).wait()
            if with_deg:
                pltpu.make_async_copy(ones_v, dacc.at[sl[1]], sl[5]).wait()

        s0, s1 = slots
        stage(s0, 0)
        fire_g(s0)
        stage(s1, 1)
        fire_g(s1)
        wait_g(s0)
        fire_s(s0)

        def pbody(p, carry):
            c0 = 2 * p + 2
            wait_s(s0)
            stage(s0, c0)
            fire_g(s0)
            wait_g(s1)
            fire_s(s1)
            wait_s(s1)
            stage(s1, c0 + 1)
            fire_g(s1)
            wait_g(s0)
            fire_s(s0)
            return carry

        lax.fori_loop(0, (NFULL - 2) // 2, pbody, 0)

        wait_g(s1)
        fire_s(s1)

        sidx_t[pl.ds(0, TAIL)] = srcall[pl.ds(NFULL * CH, TAIL)]
        didx_t[pl.ds(0, TAIL)] = dstall[pl.ds(NFULL * CH, TAIL)]
        pltpu.async_copy(tab_h.at[sidx_t], rows_t, gsem0).wait()
        wait_s(s0)
        wait_s(s1)
        pltpu.async_copy(rows_t, acc.at[didx_t], ssem0, add=True).wait()
        if with_deg:
            pltpu.async_copy(ones_v.at[pl.ds(0, TAIL)], dacc.at[didx_t],
                             dsem0, add=True).wait()

        plsc.subcore_barrier()

        pltpu.sync_copy(acc.at[pl.ds(rbase, WPT)],
                        part_h.at[c, pl.ds(rbase, WPT)])

        if with_deg:
            pltpu.sync_copy(dacc.at[pl.ds(rbase, WPT)],
                            degp_h.at[c, pl.ds(rbase, WPT)])

        @pl.when(s == NS - 1)
        def _():
            pltpu.sync_copy(acc.at[pl.ds(NS * WPT, WREM)],
                            part_h.at[c, pl.ds(NS * WPT, WREM)])
            if with_deg:
                pltpu.sync_copy(dacc.at[pl.ds(NS * WPT, WREM)],
                                degp_h.at[c, pl.ds(NS * WPT, WREM)])

    ot = tuple(out_type) if with_deg else out_type[0]
    return pl.kernel(
        body, out_type=ot, mesh=mesh, scratch_types=scratch,
        compiler_params=pltpu.CompilerParams(use_tc_tiling_on_sc=False))


_seg_sum_deg = _seg_sum_kernel(True)
_seg_sum = _seg_sum_kernel(False)


BLK = 2000


def _tc1_body(x_ref, w1l_ref, w1r_ref, b1_ref, y1_ref, z1_ref):
    xb = x_ref[...]
    y1_ref[...] = jnp.dot(xb, w1l_ref[...], preferred_element_type=jnp.float32)
    z1_ref[...] = (jnp.dot(xb, w1r_ref[...],
                           preferred_element_type=jnp.float32) + b1_ref[...])


def _tc2_body(p_ref, dp_ref, z1_ref, w2r_ref, b2_ref, h_ref, z2_ref, rd_ref):
    rdeg = 1.0 / jnp.maximum(dp_ref[0, :, 0:1] + dp_ref[1, :, 0:1], 1.0)
    h = jnp.maximum((p_ref[0] + p_ref[1]) * rdeg + z1_ref[...], 0.0)
    rd_ref[...] = rdeg
    h_ref[...] = h
    z2_ref[...] = (jnp.dot(h, w2r_ref[...],
                           preferred_element_type=jnp.float32) + b2_ref[...])


def _tc3_body(p_ref, rd_ref, z2_ref, w2l_ref, o_ref):
    a2 = (p_ref[0] + p_ref[1]) * rd_ref[...]
    o_ref[...] = (jnp.dot(a2, w2l_ref[...],
                          preferred_element_type=jnp.float32) + z2_ref[...])


def _row_spec(d):
    return pl.BlockSpec((BLK, d), lambda i: (i, 0))


def _part_spec(d):
    return pl.BlockSpec((NC, BLK, d), lambda i: (0, i, 0))


def _full_spec(r, d):
    return pl.BlockSpec((r, d), lambda i: (0, 0))


_tc1 = pl.pallas_call(
    _tc1_body,
    grid=(N // BLK,),
    in_specs=[_row_spec(D_IN), _full_spec(D_IN, D_H), _full_spec(D_IN, D_H),
              _full_spec(1, D_H)],
    out_specs=[_row_spec(D_H), _row_spec(D_H)],
    out_shape=[jax.ShapeDtypeStruct((N, D_H), jnp.float32),
               jax.ShapeDtypeStruct((N, D_H), jnp.float32)],
)

_tc2 = pl.pallas_call(
    _tc2_body,
    grid=(N // BLK,),
    in_specs=[_part_spec(D_H), _part_spec(DW), _row_spec(D_H),
              _full_spec(D_H, D_OUT), _full_spec(1, D_OUT)],
    out_specs=[_row_spec(D_H), _row_spec(D_OUT), _row_spec(1)],
    out_shape=[jax.ShapeDtypeStruct((N, D_H), jnp.float32),
               jax.ShapeDtypeStruct((N, D_OUT), jnp.float32),
               jax.ShapeDtypeStruct((N, 1), jnp.float32)],
)

_tc3 = pl.pallas_call(
    _tc3_body,
    grid=(N // BLK,),
    in_specs=[_part_spec(D_H), _row_spec(1), _row_spec(D_OUT),
              _full_spec(D_H, D_OUT)],
    out_specs=_row_spec(D_OUT),
    out_shape=jax.ShapeDtypeStruct((N, D_OUT), jnp.float32),
)


def kernel(x, edge_index, W1l, b1, W1r, W2l, b2, W2r):
    src = edge_index[0]
    dst = edge_index[1]

    y1, z1 = _tc1(x, W1l.T, W1r.T, b1.reshape(1, D_H))
    part1, degp = _seg_sum_deg(src, dst, y1)
    h, z2, rdeg = _tc2(part1, degp, z1, W2r.T, b2.reshape(1, D_OUT))
    part2 = _seg_sum(src, dst, h)
    return _tc3(part2, rdeg, z2, W2l.T)
